# baseline (speedup 1.0000x reference)
import jax
import jax.numpy as jnp
from jax import lax
from jax.experimental import pallas as pl
from jax.experimental.pallas import tpu as pltpu
from jax.experimental.pallas import tpu_sc as plsc

BATCH = 16384
HIST = 50
D = 64
NW = 32
TOTAL = BATCH * HIST
S = 512
NSU = TOTAL // S
SU_W = NSU // NW
OUT_FLAT = TOTAL * D


def _body(qt_hbm, table_hbm, out_hbm,
          idx0, idx1, rows0, rows1, tb0, tb1, tb2, tb3,
          gs0, gs1, ix0, ix1, ts0, ts1, ts2, ts3):
    c = lax.axis_index("c")
    s_ax = lax.axis_index("s")
    wid = s_ax * 2 + c
    su0 = wid * SU_W

    idxb = (idx0, idx1)
    rowsb = (rows0, rows1)
    tbufs = (tb0, tb1, tb2, tb3)
    gsem = (gs0, gs1)
    ixsem = (ix0, ix1)
    tsem = (ts0, ts1, ts2, ts3)

    lane = lax.iota(jnp.int32, 16)
    pbase = ((lane >> 3) << 12) + ((lane & 7) << 7)
    pjj = [pbase + jj * 1024 for jj in range(4)]

    def fire_gathers(p):
        for k in range(4):
            pltpu.async_copy(
                table_hbm.at[idxb[p].at[pl.ds(k * 128, 128)]],
                rowsb[p].at[pl.ds(k * 128, 128)], gsem[p])

    def drain_gathers(p):
        for k in range(4):
            pltpu.make_async_copy(
                table_hbm.at[pl.ds(0, 128)],
                rowsb[p].at[pl.ds(k * 128, 128)], gsem[p]).wait()

    def sub_unit(s, p):
        su = su0 + s
        drain_gathers(p)

        @pl.when(s + 2 < SU_W)
        def _():
            pltpu.async_copy(qt_hbm.at[su + 2], idxb[p], ixsem[p])

        o = 1 - p

        @pl.when(s + 1 < SU_W)
        def _():
            pltpu.make_async_copy(qt_hbm.at[su + 1], idxb[o], ixsem[o]).wait()
            fire_gathers(o)

        h = su // 32
        rem = su - h * 32
        base = h * 8192 + rem * 32

        for c16 in range(4):
            tb = tbufs[c16]
            tsm = tsem[c16]

            @pl.when(s > 0)
            def _():
                for _2 in range(2):
                    pltpu.make_async_copy(
                        tb.at[pl.ds(0, 4096)],
                        out_hbm.at[pl.ds(0, 4096)], tsm).wait()

            def rrblk(rb, cr, c16=c16, tb=tb):
                for jj in range(4):
                    vals = []
                    for r16 in range(16):
                        b = jj * 128 + rb * 16 + r16
                        vals.append(rowsb[p][b, pl.ds(c16 * 16, 16)])
                    for r16 in range(16):
                        plsc.store_scatter(
                            tb, [pjj[jj] + (rb * 16 + r16)], vals[r16])
                return cr

            lax.fori_loop(0, 2, lambda rb2, cr, f=rrblk: f(rb2 * 4 + 3, f(rb2 * 4 + 2, f(rb2 * 4 + 1, f(rb2 * 4, cr)))), 0)
            pltpu.async_copy(
                tb.at[pl.ds(0, 4096)],
                out_hbm.at[pl.ds((base + 2 * c16 * 1024) * 128, 4096)],
                tsm)
            pltpu.async_copy(
                tb.at[pl.ds(4096, 4096)],
                out_hbm.at[pl.ds((base + (2 * c16 + 1) * 1024) * 128, 4096)],
                tsm)

    pltpu.sync_copy(qt_hbm.at[su0], idx0)
    fire_gathers(0)
    pltpu.async_copy(qt_hbm.at[su0 + 1], idx1, ix1)

    def two_subunits(s0, carry):
        sub_unit(s0 * 2, 0)
        sub_unit(s0 * 2 + 1, 1)
        return carry

    lax.fori_loop(0, SU_W // 2, two_subunits, 0)
    for tb, tsm in ((tb0, ts0), (tb1, ts1), (tb2, ts2), (tb3, ts3)):
        for _2 in range(2):
            pltpu.make_async_copy(
                tb.at[pl.ds(0, 4096)], out_hbm.at[pl.ds(0, 4096)], tsm).wait()


def kernel(q, table):
    qt = jnp.transpose(q.astype(jnp.int32)).reshape(NSU, S)
    out1d = pl.kernel(
        _body,
        mesh=plsc.VectorSubcoreMesh(core_axis_name="c", subcore_axis_name="s"),
        out_type=jax.ShapeDtypeStruct((OUT_FLAT,), jnp.float32),
        scratch_types=[
            pltpu.VMEM((S,), jnp.int32),
            pltpu.VMEM((S,), jnp.int32),
            pltpu.VMEM((S, D), jnp.float32),
            pltpu.VMEM((S, D), jnp.float32),
            pltpu.VMEM((8192,), jnp.float32),
            pltpu.VMEM((8192,), jnp.float32),
            pltpu.VMEM((8192,), jnp.float32),
            pltpu.VMEM((8192,), jnp.float32),
            pltpu.SemaphoreType.DMA,
            pltpu.SemaphoreType.DMA,
            pltpu.SemaphoreType.DMA,
            pltpu.SemaphoreType.DMA,
            pltpu.SemaphoreType.DMA,
            pltpu.SemaphoreType.DMA,
            pltpu.SemaphoreType.DMA,
            pltpu.SemaphoreType.DMA,
        ],
        compiler_params=pltpu.CompilerParams(
            use_tc_tiling_on_sc=False, needs_layout_passes=False,
            disable_bounds_checks=True),
    )(qt, table)
    return (out1d.reshape(HIST, 8, 128, 8, 128)
            .transpose(2, 4, 0, 1, 3)
            .reshape(BATCH, HIST, D))


# speedup vs baseline: 1.5253x; 1.5253x over previous
import jax
import jax.numpy as jnp
from jax import lax
from jax.experimental import pallas as pl
from jax.experimental.pallas import tpu as pltpu
from jax.experimental.pallas import tpu_sc as plsc

BATCH = 16384
HIST = 50
D = 64
NW = 32
TOTAL = BATCH * HIST
S = 512
NSU = TOTAL // S
SU_W = NSU // NW
OUT_ROWS = TOTAL * D // 128


def _body(qt_hbm, table_hbm, out_hbm,
          idx0, idx1, rows0, rows1, tb0, tb1, tb2, tb3,
          gs0, gs1, ix0, ix1, ts0, ts1, ts2, ts3):
    c = lax.axis_index("c")
    s_ax = lax.axis_index("s")
    wid = s_ax * 2 + c
    su0 = wid * SU_W

    idxb = (idx0, idx1)
    rowsb = (rows0, rows1)
    tbufs = (tb0, tb1, tb2, tb3)
    gsem = (gs0, gs1)
    ixsem = (ix0, ix1)
    tsem = (ts0, ts1, ts2, ts3)

    lane = lax.iota(jnp.int32, 16)
    prow = ((lane >> 3) << 5) + (lane & 7)
    pjj = [prow + jj * 8 for jj in range(4)]
    zeros = jnp.zeros((16,), jnp.int32)

    def fire_gathers(p):
        for k in range(4):
            pltpu.async_copy(
                table_hbm.at[idxb[p].at[pl.ds(k * 128, 128)]],
                rowsb[p].at[pl.ds(k * 128, 128)], gsem[p])

    def drain_gathers(p):
        for k in range(4):
            pltpu.make_async_copy(
                table_hbm.at[pl.ds(0, 128)],
                rowsb[p].at[pl.ds(k * 128, 128)], gsem[p]).wait()

    def sub_unit(s, p):
        su = su0 + s
        drain_gathers(p)

        @pl.when(s + 2 < SU_W)
        def _():
            pltpu.async_copy(qt_hbm.at[su + 2], idxb[p], ixsem[p])

        o = 1 - p

        @pl.when(s + 1 < SU_W)
        def _():
            pltpu.make_async_copy(qt_hbm.at[su + 1], idxb[o], ixsem[o]).wait()
            fire_gathers(o)

        h = su // 32
        rem = su - h * 32
        base = h * 8192 + rem * 32

        for c16 in range(4):
            tb = tbufs[c16]
            tsm = tsem[c16]

            @pl.when(s > 0)
            def _():
                for _2 in range(2):
                    pltpu.make_async_copy(
                        tb.at[pl.ds(0, 32), pl.ds(0, 128)],
                        out_hbm.at[pl.ds(0, 32), :], tsm).wait()

            def rrblk(rb, cr, c16=c16, tb=tb):
                for jj in range(4):
                    vals = []
                    for r16 in range(16):
                        b = jj * 128 + rb * 16 + r16
                        vals.append(rowsb[p][b, pl.ds(c16 * 16, 16)])
                    for r16 in range(16):
                        plsc.store_scatter(
                            tb, [pjj[jj], zeros + (rb * 16 + r16)],
                            vals[r16])
                return cr

            lax.fori_loop(0, 2, lambda rb2, cr, f=rrblk: f(rb2 * 4 + 3, f(rb2 * 4 + 2, f(rb2 * 4 + 1, f(rb2 * 4, cr)))), 0)
            pltpu.async_copy(
                tb.at[pl.ds(0, 32), pl.ds(0, 128)],
                out_hbm.at[pl.ds(base + 2 * c16 * 1024, 32), :],
                tsm)
            pltpu.async_copy(
                tb.at[pl.ds(32, 32), pl.ds(0, 128)],
                out_hbm.at[pl.ds(base + (2 * c16 + 1) * 1024, 32), :],
                tsm)

    pltpu.sync_copy(qt_hbm.at[su0], idx0)
    fire_gathers(0)
    pltpu.async_copy(qt_hbm.at[su0 + 1], idx1, ix1)

    def two_subunits(s0, carry):
        sub_unit(s0 * 2, 0)
        sub_unit(s0 * 2 + 1, 1)
        return carry

    lax.fori_loop(0, SU_W // 2, two_subunits, 0)
    for tb, tsm in ((tb0, ts0), (tb1, ts1), (tb2, ts2), (tb3, ts3)):
        for _2 in range(2):
            pltpu.make_async_copy(
                tb.at[pl.ds(0, 32), pl.ds(0, 128)],
                out_hbm.at[pl.ds(0, 32), :], tsm).wait()


def kernel(q, table):
    qt = jnp.transpose(q.astype(jnp.int32)).reshape(NSU, S)
    out1d = pl.kernel(
        _body,
        mesh=plsc.VectorSubcoreMesh(core_axis_name="c", subcore_axis_name="s"),
        out_type=jax.ShapeDtypeStruct((OUT_ROWS, 128), jnp.float32),
        scratch_types=[
            pltpu.VMEM((S,), jnp.int32),
            pltpu.VMEM((S,), jnp.int32),
            pltpu.VMEM((S, D), jnp.float32),
            pltpu.VMEM((S, D), jnp.float32),
            pltpu.VMEM((64, 136), jnp.float32),
            pltpu.VMEM((64, 136), jnp.float32),
            pltpu.VMEM((64, 136), jnp.float32),
            pltpu.VMEM((64, 136), jnp.float32),
            pltpu.SemaphoreType.DMA,
            pltpu.SemaphoreType.DMA,
            pltpu.SemaphoreType.DMA,
            pltpu.SemaphoreType.DMA,
            pltpu.SemaphoreType.DMA,
            pltpu.SemaphoreType.DMA,
            pltpu.SemaphoreType.DMA,
            pltpu.SemaphoreType.DMA,
        ],
        compiler_params=pltpu.CompilerParams(
            use_tc_tiling_on_sc=False, needs_layout_passes=False,
            disable_bounds_checks=True),
    )(qt, table)
    return (out1d.reshape(HIST, 8, 128, 8, 128)
            .transpose(2, 4, 0, 1, 3)
            .reshape(BATCH, HIST, D))


# speedup vs baseline: 1.5259x; 1.0004x over previous
import jax
import jax.numpy as jnp
from jax import lax
from jax.experimental import pallas as pl
from jax.experimental.pallas import tpu as pltpu
from jax.experimental.pallas import tpu_sc as plsc

BATCH = 16384
HIST = 50
D = 64
NW = 32
TOTAL = BATCH * HIST
S = 512
NSU = TOTAL // S
SU_W = NSU // NW
OUT_ROWS = TOTAL * D // 128


def _body(qt_hbm, table_hbm, out_hbm,
          idx0, idx1, rows0, rows1, tb0, tb1, tb2, tb3,
          gs0, gs1, ix0, ix1, ts0, ts1, ts2, ts3):
    c = lax.axis_index("c")
    s_ax = lax.axis_index("s")
    wid = s_ax * 2 + c
    su0 = wid * SU_W

    idxb = (idx0, idx1)
    rowsb = (rows0, rows1)
    tbufs = (tb0, tb1, tb2, tb3)
    gsem = (gs0, gs1)
    ixsem = (ix0, ix1)
    tsem = (ts0, ts1, ts2, ts3)

    lane = lax.iota(jnp.int32, 16)
    prow = ((lane >> 3) << 5) + (lane & 7)
    pjj = [prow + jj * 8 for jj in range(4)]
    zeros = jnp.zeros((16,), jnp.int32)

    def fire_gathers(p):
        for k in range(4):
            pltpu.async_copy(
                table_hbm.at[idxb[p].at[pl.ds(k * 128, 128)]],
                rowsb[p].at[pl.ds(k * 128, 128)], gsem[p])

    def drain_gathers(p):
        pltpu.make_async_copy(
            table_hbm.at[pl.ds(0, 512)], rowsb[p], gsem[p]).wait()

    def sub_unit(s, p):
        su = su0 + s
        drain_gathers(p)

        @pl.when(s + 2 < SU_W)
        def _():
            pltpu.async_copy(qt_hbm.at[su + 2], idxb[p], ixsem[p])

        o = 1 - p

        @pl.when(s + 1 < SU_W)
        def _():
            pltpu.make_async_copy(qt_hbm.at[su + 1], idxb[o], ixsem[o]).wait()
            fire_gathers(o)

        h = su // 32
        rem = su - h * 32
        base = h * 8192 + rem * 32

        for c16 in range(4):
            tb = tbufs[c16]
            tsm = tsem[c16]

            @pl.when(s > 0)
            def _():
                pltpu.make_async_copy(
                    tb.at[pl.ds(0, 64), pl.ds(0, 128)],
                    out_hbm.at[pl.ds(0, 64), :], tsm).wait()

            def rrblk(rb, cr, c16=c16, tb=tb):
                for jj in range(4):
                    vals = []
                    for r16 in range(16):
                        b = jj * 128 + rb * 16 + r16
                        vals.append(rowsb[p][b, pl.ds(c16 * 16, 16)])
                    for r16 in range(16):
                        plsc.store_scatter(
                            tb, [pjj[jj], zeros + (rb * 16 + r16)],
                            vals[r16])
                return cr

            lax.fori_loop(0, 2, lambda rb2, cr, f=rrblk: f(rb2 * 4 + 3, f(rb2 * 4 + 2, f(rb2 * 4 + 1, f(rb2 * 4, cr)))), 0)
            pltpu.async_copy(
                tb.at[pl.ds(0, 32), pl.ds(0, 128)],
                out_hbm.at[pl.ds(base + 2 * c16 * 1024, 32), :],
                tsm)
            pltpu.async_copy(
                tb.at[pl.ds(32, 32), pl.ds(0, 128)],
                out_hbm.at[pl.ds(base + (2 * c16 + 1) * 1024, 32), :],
                tsm)

    pltpu.sync_copy(qt_hbm.at[su0], idx0)
    fire_gathers(0)
    pltpu.async_copy(qt_hbm.at[su0 + 1], idx1, ix1)

    def two_subunits(s0, carry):
        sub_unit(s0 * 2, 0)
        sub_unit(s0 * 2 + 1, 1)
        return carry

    lax.fori_loop(0, SU_W // 2, two_subunits, 0)
    for tb, tsm in ((tb0, ts0), (tb1, ts1), (tb2, ts2), (tb3, ts3)):
        pltpu.make_async_copy(
            tb.at[pl.ds(0, 64), pl.ds(0, 128)],
            out_hbm.at[pl.ds(0, 64), :], tsm).wait()


def kernel(q, table):
    qt = jnp.transpose(q.astype(jnp.int32)).reshape(NSU, S)
    out1d = pl.kernel(
        _body,
        mesh=plsc.VectorSubcoreMesh(core_axis_name="c", subcore_axis_name="s"),
        out_type=jax.ShapeDtypeStruct((OUT_ROWS, 128), jnp.float32),
        scratch_types=[
            pltpu.VMEM((S,), jnp.int32),
            pltpu.VMEM((S,), jnp.int32),
            pltpu.VMEM((S, D), jnp.float32),
            pltpu.VMEM((S, D), jnp.float32),
            pltpu.VMEM((64, 136), jnp.float32),
            pltpu.VMEM((64, 136), jnp.float32),
            pltpu.VMEM((64, 136), jnp.float32),
            pltpu.VMEM((64, 136), jnp.float32),
            pltpu.SemaphoreType.DMA,
            pltpu.SemaphoreType.DMA,
            pltpu.SemaphoreType.DMA,
            pltpu.SemaphoreType.DMA,
            pltpu.SemaphoreType.DMA,
            pltpu.SemaphoreType.DMA,
            pltpu.SemaphoreType.DMA,
            pltpu.SemaphoreType.DMA,
        ],
        compiler_params=pltpu.CompilerParams(
            use_tc_tiling_on_sc=False, needs_layout_passes=False,
            disable_bounds_checks=True),
    )(qt, table)
    return (out1d.reshape(HIST, 8, 128, 8, 128)
            .transpose(2, 4, 0, 1, 3)
            .reshape(BATCH, HIST, D))


# speedup vs baseline: 1.5263x; 1.0002x over previous
import jax
import jax.numpy as jnp
from jax import lax
from jax.experimental import pallas as pl
from jax.experimental.pallas import tpu as pltpu
from jax.experimental.pallas import tpu_sc as plsc

BATCH = 16384
HIST = 50
D = 64
NW = 32
TOTAL = BATCH * HIST
S = 512
NSU = TOTAL // S
SU_W = NSU // NW
OUT_ROWS = TOTAL * D // 128


def _body(qt_hbm, table_hbm, out_hbm,
          idx0, idx1, rows0, rows1, tb0, tb1, tb2, tb3,
          gs0, gs1, ix0, ix1, ts0, ts1, ts2, ts3):
    c = lax.axis_index("c")
    s_ax = lax.axis_index("s")
    wid = s_ax * 2 + c
    su0 = wid * SU_W

    idxb = (idx0, idx1)
    rowsb = (rows0, rows1)
    tbufs = (tb0, tb1, tb2, tb3)
    gsem = (gs0, gs1)
    ixsem = (ix0, ix1)
    tsem = (ts0, ts1, ts2, ts3)

    lane = lax.iota(jnp.int32, 16)
    prow = ((lane >> 3) << 5) + (lane & 7)
    pjj = [prow + jj * 8 for jj in range(4)]
    zeros = jnp.zeros((16,), jnp.int32)

    def fire_gathers(p):
        for k in range(4):
            pltpu.async_copy(
                table_hbm.at[idxb[p].at[pl.ds(k * 128, 128)]],
                rowsb[p].at[pl.ds(k * 128, 128)], gsem[p])

    def drain_gathers(p):
        for k in range(4):
            pltpu.make_async_copy(
                table_hbm.at[pl.ds(0, 128)],
                rowsb[p].at[pl.ds(k * 128, 128)], gsem[p]).wait()

    def sub_unit(s, p):
        su = su0 + s
        drain_gathers(p)

        @pl.when(s + 2 < SU_W)
        def _():
            pltpu.async_copy(qt_hbm.at[su + 2], idxb[p], ixsem[p])

        o = 1 - p

        @pl.when(s + 1 < SU_W)
        def _():
            pltpu.make_async_copy(qt_hbm.at[su + 1], idxb[o], ixsem[o]).wait()
            fire_gathers(o)

        h = su // 32
        rem = su - h * 32
        base = h * 8192 + rem * 32

        for c16 in range(4):
            tb = tbufs[c16]
            tsm = tsem[c16]

            @pl.when(s > 0)
            def _():
                for _2 in range(2):
                    pltpu.make_async_copy(
                        tb.at[pl.ds(0, 32), pl.ds(0, 128)],
                        out_hbm.at[pl.ds(0, 32), :], tsm).wait()

            def rrblk(rb, cr, c16=c16, tb=tb):
                for jj in range(4):
                    vals = []
                    for r16 in range(16):
                        b = jj * 128 + rb * 16 + r16
                        vals.append(rowsb[p][b, pl.ds(c16 * 16, 16)])
                    for r16 in range(16):
                        plsc.store_scatter(
                            tb, [pjj[jj], zeros + (rb * 16 + r16)],
                            vals[r16])
                return cr

            lax.fori_loop(0, 2, lambda rb2, cr, f=rrblk: f(rb2 * 4 + 3, f(rb2 * 4 + 2, f(rb2 * 4 + 1, f(rb2 * 4, cr)))), 0)
            pltpu.async_copy(
                tb.at[pl.ds(0, 32), pl.ds(0, 128)],
                out_hbm.at[pl.ds(base + 2 * c16 * 1024, 32), :],
                tsm)
            pltpu.async_copy(
                tb.at[pl.ds(32, 32), pl.ds(0, 128)],
                out_hbm.at[pl.ds(base + (2 * c16 + 1) * 1024, 32), :],
                tsm)

    pltpu.sync_copy(qt_hbm.at[su0], idx0)
    fire_gathers(0)
    pltpu.async_copy(qt_hbm.at[su0 + 1], idx1, ix1)

    def two_subunits(s0, carry):
        sub_unit(s0 * 2, 0)
        sub_unit(s0 * 2 + 1, 1)
        return carry

    lax.fori_loop(0, SU_W // 2, two_subunits, 0)
    for tb, tsm in ((tb0, ts0), (tb1, ts1), (tb2, ts2), (tb3, ts3)):
        for _2 in range(2):
            pltpu.make_async_copy(
                tb.at[pl.ds(0, 32), pl.ds(0, 128)],
                out_hbm.at[pl.ds(0, 32), :], tsm).wait()


def kernel(q, table):
    qt = jnp.transpose(q.astype(jnp.int32)).reshape(NSU, S)
    out1d = pl.kernel(
        _body,
        mesh=plsc.VectorSubcoreMesh(core_axis_name="c", subcore_axis_name="s"),
        out_type=jax.ShapeDtypeStruct((OUT_ROWS, 128), jnp.float32),
        scratch_types=[
            pltpu.VMEM((S,), jnp.int32),
            pltpu.VMEM((S,), jnp.int32),
            pltpu.VMEM((S, D), jnp.float32),
            pltpu.VMEM((S, D), jnp.float32),
            pltpu.VMEM((64, 136), jnp.float32),
            pltpu.VMEM((64, 136), jnp.float32),
            pltpu.VMEM((64, 136), jnp.float32),
            pltpu.VMEM((64, 136), jnp.float32),
            pltpu.SemaphoreType.DMA,
            pltpu.SemaphoreType.DMA,
            pltpu.SemaphoreType.DMA,
            pltpu.SemaphoreType.DMA,
            pltpu.SemaphoreType.DMA,
            pltpu.SemaphoreType.DMA,
            pltpu.SemaphoreType.DMA,
            pltpu.SemaphoreType.DMA,
        ],
        compiler_params=pltpu.CompilerParams(
            use_tc_tiling_on_sc=False, needs_layout_passes=False,
            disable_bounds_checks=True),
    )(qt, table)
    return (out1d.reshape(HIST, 8, 128, 8, 128)
            .transpose(2, 4, 0, 1, 3)
            .reshape(BATCH, HIST, D))
